# Initial kernel scaffold; baseline (speedup 1.0000x reference)
import functools

import jax
import jax.numpy as jnp
from jax import lax
from jax.experimental import pallas as pl
from jax.experimental.pallas import tpu as pltpu
from jax.experimental.pallas import tpu_sc as plsc

_NUM_CORES = 2
_NUM_SUBCORES = 16
_NW = _NUM_CORES * _NUM_SUBCORES
_CHUNK = 8


def _embed_body(table_hbm, ids_hbm, out_hbm, idx_v, buf0, buf1, sem0, sem1):
    n_ids = ids_hbm.shape[0]
    b_per_w = n_ids // _NW
    n_chunks = b_per_w // _CHUNK

    wid = lax.axis_index("s") * _NUM_CORES + lax.axis_index("c")
    base = wid * b_per_w

    pltpu.sync_copy(ids_hbm.at[pl.ds(base, b_per_w)], idx_v)

    bufs = (buf0, buf1)
    sems = (sem0, sem1)

    def start_gather(g, slot):
        pltpu.async_copy(
            table_hbm.at[idx_v.at[pl.ds(g * _CHUNK, _CHUNK)]],
            bufs[slot],
            sems[slot],
        )

    def drain(g, slot):
        pltpu.make_async_copy(
            table_hbm.at[idx_v.at[pl.ds(0, _CHUNK)]], bufs[slot], sems[slot]
        ).wait()
        pltpu.sync_copy(bufs[slot], out_hbm.at[pl.ds(base + g * _CHUNK, _CHUNK)])

    start_gather(0, 0)
    start_gather(1, 1)

    @functools.partial(pl.loop, 0, n_chunks - 2, step=2)
    def _(g0):
        for b in range(2):
            g = g0 + b
            drain(g, b)
            start_gather(g + 2, b)

    drain(n_chunks - 2, 0)
    drain(n_chunks - 1, 1)


def kernel(input_ids, table):
    batch, seq = input_ids.shape
    vocab, d = table.shape
    ids_flat = input_ids.reshape(batch * seq).astype(jnp.int32)

    mesh = plsc.VectorSubcoreMesh(
        core_axis_name="c",
        subcore_axis_name="s",
        num_cores=_NUM_CORES,
        num_subcores=_NUM_SUBCORES,
    )

    run = pl.kernel(
        _embed_body,
        out_type=jax.ShapeDtypeStruct((batch * seq, d), jnp.float32),
        mesh=mesh,
        scratch_types=[
            pltpu.VMEM(((batch * seq) // _NW,), jnp.int32),
            pltpu.VMEM((_CHUNK, d), jnp.float32),
            pltpu.VMEM((_CHUNK, d), jnp.float32),
            pltpu.SemaphoreType.DMA,
            pltpu.SemaphoreType.DMA,
        ],
    )
    out = run(table, ids_flat)
    return out.reshape(batch, seq, d)


# speedup vs baseline: 1.7716x; 1.7716x over previous
import functools

import jax
import jax.numpy as jnp
from jax import lax
from jax.experimental import pallas as pl
from jax.experimental.pallas import tpu as pltpu
from jax.experimental.pallas import tpu_sc as plsc

_NUM_CORES = 2
_NUM_SUBCORES = 16
_NW = _NUM_CORES * _NUM_SUBCORES
_CHUNK = 8


def _embed_body(table_hbm, ids_hbm, out_hbm, idx_v, buf0, buf1, sem0, sem1):
    n_ids = ids_hbm.shape[0]
    b_per_w = n_ids // _NW
    n_chunks = b_per_w // _CHUNK

    wid = lax.axis_index("s") * _NUM_CORES + lax.axis_index("c")
    base = wid * b_per_w

    pltpu.sync_copy(ids_hbm.at[pl.ds(base, b_per_w)], idx_v)

    bufs = (buf0, buf1)
    sems = (sem0, sem1)

    def start_gather(g, slot):
        pltpu.async_copy(
            table_hbm.at[idx_v.at[pl.ds(g * _CHUNK, _CHUNK)]],
            bufs[slot],
            sems[slot],
        )

    def drain(g, slot):
        pltpu.make_async_copy(
            table_hbm.at[idx_v.at[pl.ds(0, _CHUNK)]], bufs[slot], sems[slot]
        ).wait()
        pltpu.sync_copy(bufs[slot], out_hbm.at[pl.ds(base + g * _CHUNK, _CHUNK)])

    start_gather(0, 0)
    start_gather(1, 1)

    @pl.loop(0, n_chunks - 2, step=2)
    def _(g0):
        for b in range(2):
            g = g0 + b
            drain(g, b)
            start_gather(g + 2, b)

    drain(n_chunks - 2, 0)
    drain(n_chunks - 1, 1)


def kernel(input_ids, table):
    batch, seq = input_ids.shape
    vocab, d = table.shape
    ids_flat = input_ids.reshape(batch * seq).astype(jnp.int32)

    mesh = plsc.VectorSubcoreMesh(
        core_axis_name="c",
        subcore_axis_name="s",
        num_cores=_NUM_CORES,
        num_subcores=_NUM_SUBCORES,
    )

    run = pl.kernel(
        _embed_body,
        out_type=jax.ShapeDtypeStruct((batch * seq, d), jnp.float32),
        mesh=mesh,
        scratch_types=[
            pltpu.VMEM(((batch * seq) // _NW,), jnp.int32),
            pltpu.VMEM((_CHUNK, d), jnp.float32),
            pltpu.VMEM((_CHUNK, d), jnp.float32),
            pltpu.SemaphoreType.DMA,
            pltpu.SemaphoreType.DMA,
        ],
    )
    out = run(table, ids_flat)
    return out.reshape(batch, seq, d)
